# Initial kernel scaffold; baseline (speedup 1.0000x reference)
import jax
import jax.numpy as jnp
from jax.experimental import pallas as pl


def kernel(cat_field, port_field, num_field, ip_field, mac_field, cat_table, port_table, num_w, num_b, ip_tables, ip_conv_w, ip_conv_b, mac_tables, mac_conv_w, mac_conv_b):
    raise NotImplementedError("write your pallas kernel here")


# speedup vs baseline: 1.8239x; 1.8239x over previous
import functools

import jax
import jax.numpy as jnp
from jax import lax
from jax.experimental import pallas as pl
from jax.experimental.pallas import tpu as pltpu
from jax.experimental.pallas import tpu_sc as plsc

B = 16384
CAT_DIM = 64
PORT_DIM = 16
NUM_DIM = 16
OCTET_DIM = 32


_NUM_CORES = 2
_NUM_SUBCORES = 16


@functools.cache
def _make_cat_gather():
    nw = _NUM_CORES * _NUM_SUBCORES
    b_per_w = B // nw
    mesh = plsc.VectorSubcoreMesh(core_axis_name="c", subcore_axis_name="s")

    @functools.partial(
        pl.kernel,
        mesh=mesh,
        out_type=jax.ShapeDtypeStruct((B, CAT_DIM), jnp.float32),
        scratch_types=[
            pltpu.VMEM((b_per_w,), jnp.int32),
            pltpu.VMEM((b_per_w, CAT_DIM), jnp.float32),
            pltpu.SemaphoreType.DMA,
        ],
    )
    def cat_gather(idx_hbm, table_hbm, out_hbm, idx_v, rows_v, sem):
        wid = lax.axis_index("s") * _NUM_CORES + lax.axis_index("c")
        base = wid * b_per_w
        pltpu.sync_copy(idx_hbm.at[pl.ds(base, b_per_w)], idx_v)
        pltpu.async_copy(table_hbm.at[idx_v], rows_v, sem).wait()
        pltpu.sync_copy(rows_v, out_hbm.at[pl.ds(base, b_per_w)])

    return cat_gather


_BLK = 512
_GRID = B // _BLK


def _heads_body(port_ref, num_ref, ip_ref, mac_ref,
                port_tab_ref, num_w_ref, num_b_ref,
                ip_tab_ref, ip_w_ref, ip_b_ref,
                mac_tab_ref, mac_w_ref, mac_b_ref,
                port_out_ref, num_out_ref, ip_out_ref, mac_out_ref,
                fold_ip_ref, fold_mac_ref):
    @pl.when(pl.program_id(0) == 0)
    def _fold():
        for (w_ref, tab_ref, fold_ref, n_oct) in (
                (ip_w_ref, ip_tab_ref, fold_ip_ref, 4),
                (mac_w_ref, mac_tab_ref, fold_mac_ref, 6)):
            w0 = w_ref[0]
            w1 = w_ref[1]
            w2 = w_ref[2]
            a_first = w0 + w1
            a_mid = w0 + w1 + w2
            a_last = w1 + w2
            scale = 1.0 / n_oct
            for j in range(n_oct):
                a = a_first if j == 0 else (a_last if j == n_oct - 1 else a_mid)
                f = lax.dot_general(tab_ref[j], a, (((1,), (1,)), ((), ())),
                                    preferred_element_type=jnp.float32)
                fold_ref[j * 256:(j + 1) * 256, :] = f * scale

    f32 = jnp.float32
    pb = port_ref[0, 0, :]
    oh4 = (pb[:, None] == lax.broadcasted_iota(jnp.int32, (_BLK, 4), 1))
    port_out_ref[...] = jnp.dot(oh4.astype(f32), port_tab_ref[...],
                                preferred_element_type=f32)
    nb = num_ref[0, 0, :]
    num_out_ref[...] = nb[:, None] * num_w_ref[0, :][None, :] + num_b_ref[0, :][None, :]

    iota256 = lax.broadcasted_iota(jnp.int32, (_BLK, 256), 1)
    for (idx_ref, fold_ref, b_ref, out_ref, n_oct) in (
            (ip_ref, fold_ip_ref, ip_b_ref, ip_out_ref, 4),
            (mac_ref, fold_mac_ref, mac_b_ref, mac_out_ref, 6)):
        idx = idx_ref[0]
        acc = jnp.broadcast_to(b_ref[0, :][None, :], (_BLK, OCTET_DIM))
        for j in range(n_oct):
            oh = (idx[:, j:j + 1] == iota256).astype(f32)
            acc = acc + jnp.dot(oh, fold_ref[j * 256:(j + 1) * 256, :],
                                preferred_element_type=f32)
        out_ref[...] = acc


def _whole(shape):
    nd = len(shape)
    return pl.BlockSpec(shape, lambda i, _nd=nd: (0,) * _nd)


_HEADS_IN_SPECS = [
        pl.BlockSpec((1, 1, _BLK), lambda i: (i, 0, 0)),
        pl.BlockSpec((1, 1, _BLK), lambda i: (i, 0, 0)),
        pl.BlockSpec((1, _BLK, 4), lambda i: (i, 0, 0)),
        pl.BlockSpec((1, _BLK, 6), lambda i: (i, 0, 0)),
        _whole((4, PORT_DIM)),
        _whole((1, NUM_DIM)),
        _whole((1, NUM_DIM)),
        _whole((4, 256, OCTET_DIM)),
        _whole((3, OCTET_DIM, OCTET_DIM)),
        _whole((1, OCTET_DIM)),
        _whole((6, 256, OCTET_DIM)),
        _whole((3, OCTET_DIM, OCTET_DIM)),
        _whole((1, OCTET_DIM)),
]
_HEADS_OUT_SPECS = [
    pl.BlockSpec((_BLK, PORT_DIM), lambda i: (i, 0)),
    pl.BlockSpec((_BLK, NUM_DIM), lambda i: (i, 0)),
    pl.BlockSpec((_BLK, OCTET_DIM), lambda i: (i, 0)),
    pl.BlockSpec((_BLK, OCTET_DIM), lambda i: (i, 0)),
]
_HEADS_OUT_SHAPE = [
    jax.ShapeDtypeStruct((B, PORT_DIM), jnp.float32),
    jax.ShapeDtypeStruct((B, NUM_DIM), jnp.float32),
    jax.ShapeDtypeStruct((B, OCTET_DIM), jnp.float32),
    jax.ShapeDtypeStruct((B, OCTET_DIM), jnp.float32),
]
_HEADS_SCRATCH = [
    pltpu.VMEM((4 * 256, OCTET_DIM), jnp.float32),
    pltpu.VMEM((6 * 256, OCTET_DIM), jnp.float32),
]

_heads_call = pl.pallas_call(
    _heads_body,
    grid=(_GRID,),
    in_specs=_HEADS_IN_SPECS,
    out_specs=_HEADS_OUT_SPECS,
    out_shape=_HEADS_OUT_SHAPE,
    scratch_shapes=_HEADS_SCRATCH,
)


def kernel(cat_field, port_field, num_field, ip_field, mac_field,
           cat_table, port_table, num_w, num_b,
           ip_tables, ip_conv_w, ip_conv_b,
           mac_tables, mac_conv_w, mac_conv_b):
    cat_emb = jnp.take(cat_table, cat_field, axis=0)
    port_emb, num_emb, ip_emb, mac_emb = _heads_call(
        port_field.reshape(_GRID, 1, _BLK),
        num_field.reshape(_GRID, 1, _BLK),
        ip_field.reshape(_GRID, _BLK, 4),
        mac_field.reshape(_GRID, _BLK, 6),
        port_table,
        num_w.reshape(1, NUM_DIM),
        num_b.reshape(1, NUM_DIM),
        ip_tables,
        jnp.transpose(ip_conv_w, (2, 0, 1)),
        ip_conv_b.reshape(1, OCTET_DIM),
        mac_tables,
        jnp.transpose(mac_conv_w, (2, 0, 1)),
        mac_conv_b.reshape(1, OCTET_DIM),
    )
    return (cat_emb, port_emb, num_emb, ip_emb, mac_emb)
